# Initial kernel scaffold; baseline (speedup 1.0000x reference)
import jax
import jax.numpy as jnp
from jax.experimental import pallas as pl


def kernel(features, edges, W1, att_src1, att_dst1, b1, W2, att_src2, att_dst2, b2):
    raise NotImplementedError("write your pallas kernel here")


# speedup vs baseline: 26.7477x; 26.7477x over previous
import functools

import jax
import jax.numpy as jnp
from jax import lax
from jax.experimental import pallas as pl
from jax.experimental.pallas import tpu as pltpu, tpu_sc as plsc

N = 10000
NP = 10240
F_IN = 128
HID = 128
CLS = 64
E = 320000
E2 = E + N
NW = 32
BLK = 128
NB = 81
EPT = NB * BLK
EP = NW * EPT
RPS = NP // 16


def _dense1_body(x_ref, w_ref, asrc_ref, adst_ref, h_ref, as_ref, ad_ref):
    h = jnp.dot(x_ref[...], w_ref[...], preferred_element_type=jnp.float32)
    h_ref[...] = h
    as_ref[...] = jnp.sum(h * asrc_ref[...][None, :], axis=1, keepdims=True)
    ad_ref[...] = jnp.sum(h * adst_ref[...][None, :], axis=1, keepdims=True)


def _combine2_body(part_ref, s_ref, b_ref, w2_ref, asrc_ref, adst_ref,
                   h2_ref, as_ref, ad_ref):
    p = part_ref[0] + part_ref[1]
    s = s_ref[0] + s_ref[1]
    inv = 1.0 / (s + 1e-16)
    x2 = jnp.maximum(p * jnp.reshape(inv, (NP, 1)) + b_ref[...][None, :], 0.0)
    h2 = jnp.dot(x2, w2_ref[...], preferred_element_type=jnp.float32)
    h2_ref[...] = h2
    as_ref[...] = jnp.sum(h2 * asrc_ref[...][None, :], axis=1, keepdims=True)
    ad_ref[...] = jnp.sum(h2 * adst_ref[...][None, :], axis=1, keepdims=True)


def _final_body(part_ref, s_ref, b_ref, out_ref):
    p = part_ref[0] + part_ref[1]
    s = s_ref[0] + s_ref[1]
    inv = 1.0 / (s + 1e-16)
    z = p * jnp.reshape(inv, (NP, 1)) + b_ref[...][None, :]
    m = jnp.max(z, axis=1, keepdims=True)
    lse = jnp.log(jnp.sum(jnp.exp(z - m), axis=1, keepdims=True)) + m
    out_ref[...] = z - lse


@functools.lru_cache(maxsize=None)
def _make_sc_edge(D):
    mesh = plsc.VectorSubcoreMesh(core_axis_name="c", subcore_axis_name="s",
                                  num_cores=2, num_subcores=16)

    def body(src3, dst3, asrc_hbm, adst_hbm, h_hbm, part_out, s_out,
             srcj, dstj, asrc_v, adst_v, exb, rowsb, sem, acc_sh, s_sh):
        c = lax.axis_index("c")
        s = lax.axis_index("s")
        wid = s * 2 + c
        lane = lax.broadcasted_iota(jnp.int32, (16,), 0)

        pltpu.sync_copy(asrc_hbm, asrc_v)
        pltpu.sync_copy(adst_hbm, adst_v)

        def _zb_zero(r, _):
            for v in range(D // 16):
                rowsb[r, pl.ds(v * 16, 16)] = jnp.zeros((16,), jnp.float32)
            return 0
        lax.fori_loop(0, BLK, _zb_zero, 0)
        for i in range(BLK // 16):
            exb[pl.ds(i * 16, 16)] = jnp.zeros((16,), jnp.float32)
        for k in range(RPS // BLK):
            pltpu.sync_copy(rowsb, acc_sh.at[pl.ds(s * RPS + k * BLK, BLK)])
            pltpu.sync_copy(exb, s_sh.at[pl.ds(s * RPS + k * BLK, BLK)])
        plsc.subcore_barrier()

        def block(j, _):
            pltpu.sync_copy(src3.at[wid, j], srcj)
            pltpu.sync_copy(dst3.at[wid, j], dstj)
            gather = pltpu.async_copy(h_hbm.at[srcj], rowsb, sem)
            base = wid * EPT + j * BLK
            for i in range(BLK // 16):
                sidx = srcj[pl.ds(i * 16, 16)]
                didx = dstj[pl.ds(i * 16, 16)]
                a_s = plsc.load_gather(asrc_v, [sidx])
                a_d = plsc.load_gather(adst_v, [didx])
                t = a_s + a_d
                e = jnp.where(t > 0, t, 0.2 * t)
                ex = jnp.exp(e)
                gid = base + i * 16 + lane
                ex = jnp.where(gid < E2, ex, 0.0)
                exb[pl.ds(i * 16, 16)] = ex
            gather.wait()

            def scale(i, _):
                m = plsc.load_gather(exb, [jnp.full((16,), i, jnp.int32)])
                for v in range(D // 16):
                    sl = pl.ds(v * 16, 16)
                    rowsb[i, sl] = rowsb[i, sl] * m
                return 0
            lax.fori_loop(0, BLK, scale, 0)

            pltpu.sync_copy(rowsb, acc_sh.at[dstj], add=True)
            pltpu.sync_copy(exb, s_sh.at[dstj], add=True)
            return 0

        lax.fori_loop(0, NB, block, 0)
        plsc.subcore_barrier()

        off = s * RPS
        pltpu.sync_copy(acc_sh.at[pl.ds(off, RPS)], part_out.at[c, pl.ds(off, RPS)])
        pltpu.sync_copy(s_sh.at[pl.ds(off, RPS)], s_out.at[c, pl.ds(off, RPS)])

    return pl.kernel(
        body,
        out_type=[
            jax.ShapeDtypeStruct((2, NP, D), jnp.float32),
            jax.ShapeDtypeStruct((2, NP), jnp.float32),
        ],
        mesh=mesh,
        scratch_types=[
            pltpu.VMEM((BLK,), jnp.int32),
            pltpu.VMEM((BLK,), jnp.int32),
            pltpu.VMEM((NP,), jnp.float32),
            pltpu.VMEM((NP,), jnp.float32),
            pltpu.VMEM((BLK,), jnp.float32),
            pltpu.VMEM((BLK, D), jnp.float32),
            pltpu.SemaphoreType.DMA,
            pltpu.VMEM_SHARED((NP, D), jnp.float32),
            pltpu.VMEM_SHARED((NP,), jnp.float32),
        ],
        compiler_params=pltpu.CompilerParams(
            needs_layout_passes=False, use_tc_tiling_on_sc=False),
    )


_dense1 = pl.pallas_call(
    _dense1_body,
    out_shape=[
        jax.ShapeDtypeStruct((NP, HID), jnp.float32),
        jax.ShapeDtypeStruct((NP, 1), jnp.float32),
        jax.ShapeDtypeStruct((NP, 1), jnp.float32),
    ],
)

_combine2 = pl.pallas_call(
    _combine2_body,
    out_shape=[
        jax.ShapeDtypeStruct((NP, CLS), jnp.float32),
        jax.ShapeDtypeStruct((NP, 1), jnp.float32),
        jax.ShapeDtypeStruct((NP, 1), jnp.float32),
    ],
)

_final = pl.pallas_call(
    _final_body,
    out_shape=jax.ShapeDtypeStruct((NP, CLS), jnp.float32),
)


def kernel(features, edges, W1, att_src1, att_dst1, b1, W2, att_src2, att_dst2, b2):
    loop = jnp.arange(N, dtype=edges.dtype)
    src = jnp.concatenate([edges[0], loop, jnp.zeros((EP - E2,), edges.dtype)])
    dst = jnp.concatenate([edges[1], loop, jnp.zeros((EP - E2,), edges.dtype)])
    src3 = src.reshape(NW, NB, BLK)
    dst3 = dst.reshape(NW, NB, BLK)

    xp = jnp.pad(features, ((0, NP - N), (0, 0)))

    h1, a1s, a1d = _dense1(xp, W1, att_src1, att_dst1)
    part1, s1 = _make_sc_edge(HID)(src3, dst3, a1s.reshape(NP), a1d.reshape(NP), h1)
    h2, a2s, a2d = _combine2(part1, s1, b1, W2, att_src2, att_dst2)
    part2, s2 = _make_sc_edge(CLS)(src3, dst3, a2s.reshape(NP), a2d.reshape(NP), h2)
    out = _final(part2, s2, b2)
    return out[:N]


# speedup vs baseline: 27.6338x; 1.0331x over previous
import functools

import jax
import jax.numpy as jnp
from jax import lax
from jax.experimental import pallas as pl
from jax.experimental.pallas import tpu as pltpu, tpu_sc as plsc

N = 10000
NP = 10240
F_IN = 128
HID = 128
CLS = 64
E = 320000
E2 = E + N
NW = 32
BLK = 128
NB = 82
EPT = NB * BLK
EP = NW * EPT
RPS = NP // 16


def _dense1_body(x_ref, w_ref, asrc_ref, adst_ref, h_ref, as_ref, ad_ref):
    h = jnp.dot(x_ref[...], w_ref[...], preferred_element_type=jnp.float32)
    h_ref[...] = h
    as_ref[...] = jnp.sum(h * asrc_ref[...][None, :], axis=1, keepdims=True)
    ad_ref[...] = jnp.sum(h * adst_ref[...][None, :], axis=1, keepdims=True)


def _combine2_body(part_ref, s_ref, b_ref, w2_ref, asrc_ref, adst_ref,
                   h2_ref, as_ref, ad_ref):
    p = part_ref[0] + part_ref[1]
    s = s_ref[0] + s_ref[1]
    inv = 1.0 / (s + 1e-16)
    x2 = jnp.maximum(p * jnp.reshape(inv, (NP, 1)) + b_ref[...][None, :], 0.0)
    h2 = jnp.dot(x2, w2_ref[...], preferred_element_type=jnp.float32)
    h2_ref[...] = h2
    as_ref[...] = jnp.sum(h2 * asrc_ref[...][None, :], axis=1, keepdims=True)
    ad_ref[...] = jnp.sum(h2 * adst_ref[...][None, :], axis=1, keepdims=True)


def _final_body(part_ref, s_ref, b_ref, out_ref):
    p = part_ref[0] + part_ref[1]
    s = s_ref[0] + s_ref[1]
    inv = 1.0 / (s + 1e-16)
    z = p * jnp.reshape(inv, (NP, 1)) + b_ref[...][None, :]
    m = jnp.max(z, axis=1, keepdims=True)
    lse = jnp.log(jnp.sum(jnp.exp(z - m), axis=1, keepdims=True)) + m
    out_ref[...] = z - lse


@functools.lru_cache(maxsize=None)
def _make_sc_edge(D):
    mesh = plsc.VectorSubcoreMesh(core_axis_name="c", subcore_axis_name="s",
                                  num_cores=2, num_subcores=16)

    def body(edges4, asrc_hbm, adst_hbm, h_hbm, part_out, s_out,
             idxb0, idxb1, asb0, asb1, adb0, adb1, exb0, exb1, rows0, rows1,
             absem0, absem1, grsem0, grsem1, rsem0, rsem1, ssem0, ssem1,
             asrc_sh, adst_sh, acc_sh, s_sh):
        idxb = (idxb0, idxb1)
        asb = (asb0, asb1)
        adb = (adb0, adb1)
        exb = (exb0, exb1)
        rows = (rows0, rows1)
        absem = (absem0, absem1)
        grsem = (grsem0, grsem1)
        rsem = (rsem0, rsem1)
        ssem = (ssem0, ssem1)

        c = lax.axis_index("c")
        s = lax.axis_index("s")
        wid = s * 2 + c
        lane = lax.broadcasted_iota(jnp.int32, (16,), 0)

        def _zb_zero(r, _):
            for v in range(D // 16):
                rows0[r, pl.ds(v * 16, 16)] = jnp.zeros((16,), jnp.float32)
            return 0
        lax.fori_loop(0, BLK, _zb_zero, 0)
        for i in range(BLK // 16):
            exb0[pl.ds(i * 16, 16)] = jnp.zeros((16,), jnp.float32)
        for k in range(RPS // BLK):
            pltpu.sync_copy(rows0, acc_sh.at[pl.ds(s * RPS + k * BLK, BLK)])
            pltpu.sync_copy(exb0, s_sh.at[pl.ds(s * RPS + k * BLK, BLK)])
        @pl.when(s == 0)
        def _():
            pltpu.sync_copy(asrc_hbm, asrc_sh)
            pltpu.sync_copy(adst_hbm, adst_sh)
        plsc.subcore_barrier()

        def prefetch(j, b):
            pltpu.sync_copy(edges4.at[wid, j], idxb[b])
            pltpu.async_copy(asrc_sh.at[idxb[b].at[0]], asb[b], absem[b])
            pltpu.async_copy(adst_sh.at[idxb[b].at[1]], adb[b], absem[b])
            pltpu.async_copy(h_hbm.at[idxb[b].at[0]], rows[b], grsem[b])

        prefetch(0, 0)

        def step(j, b, nb):
            pltpu.make_async_copy(asrc_sh.at[idxb[b].at[0]], asb[b], absem[b]).wait()
            pltpu.make_async_copy(adst_sh.at[idxb[b].at[1]], adb[b], absem[b]).wait()
            base = wid * EPT + j * BLK
            for i in range(BLK // 16):
                t = asb[b][pl.ds(i * 16, 16)] + adb[b][pl.ds(i * 16, 16)]
                e = jnp.where(t > 0, t, 0.2 * t)
                ex = jnp.exp(e)
                gid = base + i * 16 + lane
                ex = jnp.where(gid < E2, ex, 0.0)
                exb[b][pl.ds(i * 16, 16)] = ex
            pltpu.async_copy(exb[b], s_sh.at[idxb[b].at[1]], ssem[b], add=True)

            @pl.when(j + 1 < NB)
            def _():
                @pl.when(j >= 1)
                def _():
                    pltpu.make_async_copy(
                        rows[nb], acc_sh.at[idxb[nb].at[1]], rsem[nb]).wait()
                    pltpu.make_async_copy(
                        exb[nb], s_sh.at[idxb[nb].at[1]], ssem[nb]).wait()
                prefetch(j + 1, nb)

            pltpu.make_async_copy(h_hbm.at[idxb[b].at[0]], rows[b], grsem[b]).wait()

            @plsc.parallel_loop(0, BLK, step=1, unroll=8)
            def _(i):
                m = plsc.load_gather(exb[b], [jnp.full((16,), i, jnp.int32)])
                for v in range(D // 16):
                    sl = pl.ds(v * 16, 16)
                    rows[b][i, sl] = rows[b][i, sl] * m

            pltpu.async_copy(rows[b], acc_sh.at[idxb[b].at[1]], rsem[b], add=True)

        def outer(g, _):
            step(2 * g, 0, 1)
            step(2 * g + 1, 1, 0)
            return 0
        lax.fori_loop(0, NB // 2, outer, 0)

        for b in (0, 1):
            pltpu.make_async_copy(rows[b], acc_sh.at[idxb[b].at[1]], rsem[b]).wait()
            pltpu.make_async_copy(exb[b], s_sh.at[idxb[b].at[1]], ssem[b]).wait()
        plsc.subcore_barrier()

        off = s * RPS
        pltpu.sync_copy(acc_sh.at[pl.ds(off, RPS)], part_out.at[c, pl.ds(off, RPS)])
        pltpu.sync_copy(s_sh.at[pl.ds(off, RPS)], s_out.at[c, pl.ds(off, RPS)])

    return pl.kernel(
        body,
        out_type=[
            jax.ShapeDtypeStruct((2, NP, D), jnp.float32),
            jax.ShapeDtypeStruct((2, NP), jnp.float32),
        ],
        mesh=mesh,
        scratch_types=[
            pltpu.VMEM((2, BLK), jnp.int32),
            pltpu.VMEM((2, BLK), jnp.int32),
            pltpu.VMEM((BLK,), jnp.float32),
            pltpu.VMEM((BLK,), jnp.float32),
            pltpu.VMEM((BLK,), jnp.float32),
            pltpu.VMEM((BLK,), jnp.float32),
            pltpu.VMEM((BLK,), jnp.float32),
            pltpu.VMEM((BLK,), jnp.float32),
            pltpu.VMEM((BLK, D), jnp.float32),
            pltpu.VMEM((BLK, D), jnp.float32),
            pltpu.SemaphoreType.DMA,
            pltpu.SemaphoreType.DMA,
            pltpu.SemaphoreType.DMA,
            pltpu.SemaphoreType.DMA,
            pltpu.SemaphoreType.DMA,
            pltpu.SemaphoreType.DMA,
            pltpu.SemaphoreType.DMA,
            pltpu.SemaphoreType.DMA,
            pltpu.VMEM_SHARED((NP,), jnp.float32),
            pltpu.VMEM_SHARED((NP,), jnp.float32),
            pltpu.VMEM_SHARED((NP, D), jnp.float32),
            pltpu.VMEM_SHARED((NP,), jnp.float32),
        ],
        compiler_params=pltpu.CompilerParams(
            needs_layout_passes=False, use_tc_tiling_on_sc=False),
    )


_dense1 = pl.pallas_call(
    _dense1_body,
    out_shape=[
        jax.ShapeDtypeStruct((NP, HID), jnp.float32),
        jax.ShapeDtypeStruct((NP, 1), jnp.float32),
        jax.ShapeDtypeStruct((NP, 1), jnp.float32),
    ],
)

_combine2 = pl.pallas_call(
    _combine2_body,
    out_shape=[
        jax.ShapeDtypeStruct((NP, CLS), jnp.float32),
        jax.ShapeDtypeStruct((NP, 1), jnp.float32),
        jax.ShapeDtypeStruct((NP, 1), jnp.float32),
    ],
)

_final = pl.pallas_call(
    _final_body,
    out_shape=jax.ShapeDtypeStruct((NP, CLS), jnp.float32),
)


def kernel(features, edges, W1, att_src1, att_dst1, b1, W2, att_src2, att_dst2, b2):
    loop = jnp.arange(N, dtype=edges.dtype)
    src = jnp.concatenate([edges[0], loop, jnp.zeros((EP - E2,), edges.dtype)])
    dst = jnp.concatenate([edges[1], loop, jnp.zeros((EP - E2,), edges.dtype)])
    edges4 = jnp.stack([src.reshape(NW, NB, BLK), dst.reshape(NW, NB, BLK)], axis=2)

    xp = jnp.pad(features, ((0, NP - N), (0, 0)))

    h1, a1s, a1d = _dense1(xp, W1, att_src1, att_dst1)
    part1, s1 = _make_sc_edge(HID)(edges4, a1s.reshape(NP), a1d.reshape(NP), h1)
    h2, a2s, a2d = _combine2(part1, s1, b1, W2, att_src2, att_dst2)
    part2, s2 = _make_sc_edge(CLS)(edges4, a2s.reshape(NP), a2d.reshape(NP), h2)
    out = _final(part2, s2, b2)
    return out[:N]


# speedup vs baseline: 31.3262x; 1.1336x over previous
import functools

import jax
import jax.numpy as jnp
from jax import lax
from jax.experimental import pallas as pl
from jax.experimental.pallas import tpu as pltpu, tpu_sc as plsc

N = 10000
NP = 10240
F_IN = 128
HID = 128
CLS = 64
E = 320000
E2 = E + N
NW = 32
BLK = 128
NB = 82
EPT = NB * BLK
EP = NW * EPT
RPS = NP // 16


def _dense1_body(x_ref, w_ref, asrc_ref, adst_ref, h_ref, as_ref, ad_ref):
    h = jnp.dot(x_ref[...], w_ref[...], preferred_element_type=jnp.float32)
    h_ref[...] = h
    as_ref[...] = jnp.sum(h * asrc_ref[...][None, :], axis=1, keepdims=True)
    ad_ref[...] = jnp.sum(h * adst_ref[...][None, :], axis=1, keepdims=True)


def _combine2_body(part_ref, s_ref, b_ref, w2_ref, asrc_ref, adst_ref,
                   h2_ref, as_ref, ad_ref):
    p = part_ref[0] + part_ref[1]
    s = s_ref[0] + s_ref[1]
    inv = 1.0 / (s + 1e-16)
    x2 = jnp.maximum(p * jnp.reshape(inv, (NP, 1)) + b_ref[...][None, :], 0.0)
    h2 = jnp.dot(x2, w2_ref[...], preferred_element_type=jnp.float32)
    h2_ref[...] = h2
    as_ref[...] = jnp.sum(h2 * asrc_ref[...][None, :], axis=1, keepdims=True)
    ad_ref[...] = jnp.sum(h2 * adst_ref[...][None, :], axis=1, keepdims=True)


def _final_body(part_ref, s_ref, b_ref, out_ref):
    p = part_ref[0] + part_ref[1]
    s = s_ref[0] + s_ref[1]
    inv = 1.0 / (s + 1e-16)
    z = p * jnp.reshape(inv, (NP, 1)) + b_ref[...][None, :]
    m = jnp.max(z, axis=1, keepdims=True)
    lse = jnp.log(jnp.sum(jnp.exp(z - m), axis=1, keepdims=True)) + m
    out_ref[...] = z - lse


@functools.lru_cache(maxsize=None)
def _make_sc_edge(D, rows_in_spmem=False):
    mesh = plsc.VectorSubcoreMesh(core_axis_name="c", subcore_axis_name="s",
                                  num_cores=2, num_subcores=16)

    def body(edges4, asrc_hbm, adst_hbm, h_hbm, part_out, s_out,
             idxb0, idxb1, asb0, asb1, adb0, adb1, exb0, exb1, rows0, rows1,
             absem0, absem1, grsem0, grsem1, rsem0, rsem1, ssem0, ssem1,
             asrc_sh, adst_sh, acc_sh, s_sh, *maybe_h_sh):
        h_src = maybe_h_sh[0] if rows_in_spmem else h_hbm
        idxb = (idxb0, idxb1)
        asb = (asb0, asb1)
        adb = (adb0, adb1)
        exb = (exb0, exb1)
        rows = (rows0, rows1)
        absem = (absem0, absem1)
        grsem = (grsem0, grsem1)
        rsem = (rsem0, rsem1)
        ssem = (ssem0, ssem1)

        c = lax.axis_index("c")
        s = lax.axis_index("s")
        wid = s * 2 + c
        lane = lax.broadcasted_iota(jnp.int32, (16,), 0)

        def _zb_zero(r, _):
            for v in range(D // 16):
                rows0[r, pl.ds(v * 16, 16)] = jnp.zeros((16,), jnp.float32)
            return 0
        lax.fori_loop(0, BLK, _zb_zero, 0)
        for i in range(BLK // 16):
            exb0[pl.ds(i * 16, 16)] = jnp.zeros((16,), jnp.float32)
        for k in range(RPS // BLK):
            pltpu.sync_copy(rows0, acc_sh.at[pl.ds(s * RPS + k * BLK, BLK)])
            pltpu.sync_copy(exb0, s_sh.at[pl.ds(s * RPS + k * BLK, BLK)])
        @pl.when(s == 0)
        def _():
            pltpu.sync_copy(asrc_hbm, asrc_sh)
            pltpu.sync_copy(adst_hbm, adst_sh)
        if rows_in_spmem:
            pltpu.sync_copy(h_hbm.at[pl.ds(s * RPS, RPS)],
                            maybe_h_sh[0].at[pl.ds(s * RPS, RPS)])
        plsc.subcore_barrier()

        def prefetch(j, b):
            pltpu.sync_copy(edges4.at[wid, j], idxb[b])
            pltpu.async_copy(asrc_sh.at[idxb[b].at[0]], asb[b], absem[b])
            pltpu.async_copy(adst_sh.at[idxb[b].at[1]], adb[b], absem[b])
            pltpu.async_copy(h_src.at[idxb[b].at[0]], rows[b], grsem[b])

        prefetch(0, 0)

        def step(j, b, nb):
            pltpu.make_async_copy(asrc_sh.at[idxb[b].at[0]], asb[b], absem[b]).wait()
            pltpu.make_async_copy(adst_sh.at[idxb[b].at[1]], adb[b], absem[b]).wait()
            base = wid * EPT + j * BLK
            for i in range(BLK // 16):
                t = asb[b][pl.ds(i * 16, 16)] + adb[b][pl.ds(i * 16, 16)]
                e = jnp.where(t > 0, t, 0.2 * t)
                ex = jnp.exp(e)
                gid = base + i * 16 + lane
                ex = jnp.where(gid < E2, ex, 0.0)
                exb[b][pl.ds(i * 16, 16)] = ex
            pltpu.async_copy(exb[b], s_sh.at[idxb[b].at[1]], ssem[b], add=True)

            @pl.when(j + 1 < NB)
            def _():
                @pl.when(j >= 1)
                def _():
                    pltpu.make_async_copy(
                        rows[nb], acc_sh.at[idxb[nb].at[1]], rsem[nb]).wait()
                    pltpu.make_async_copy(
                        exb[nb], s_sh.at[idxb[nb].at[1]], ssem[nb]).wait()
                prefetch(j + 1, nb)

            pltpu.make_async_copy(h_src.at[idxb[b].at[0]], rows[b], grsem[b]).wait()

            @plsc.parallel_loop(0, BLK, step=1, unroll=8)
            def _(i):
                m = plsc.load_gather(exb[b], [jnp.full((16,), i, jnp.int32)])
                for v in range(D // 16):
                    sl = pl.ds(v * 16, 16)
                    rows[b][i, sl] = rows[b][i, sl] * m

            pltpu.async_copy(rows[b], acc_sh.at[idxb[b].at[1]], rsem[b], add=True)

        def outer(g, _):
            step(2 * g, 0, 1)
            step(2 * g + 1, 1, 0)
            return 0
        lax.fori_loop(0, NB // 2, outer, 0)

        for b in (0, 1):
            pltpu.make_async_copy(rows[b], acc_sh.at[idxb[b].at[1]], rsem[b]).wait()
            pltpu.make_async_copy(exb[b], s_sh.at[idxb[b].at[1]], ssem[b]).wait()
        plsc.subcore_barrier()

        off = s * RPS
        pltpu.sync_copy(acc_sh.at[pl.ds(off, RPS)], part_out.at[c, pl.ds(off, RPS)])
        pltpu.sync_copy(s_sh.at[pl.ds(off, RPS)], s_out.at[c, pl.ds(off, RPS)])

    return pl.kernel(
        body,
        out_type=[
            jax.ShapeDtypeStruct((2, NP, D), jnp.float32),
            jax.ShapeDtypeStruct((2, NP), jnp.float32),
        ],
        mesh=mesh,
        scratch_types=[
            pltpu.VMEM((2, BLK), jnp.int32),
            pltpu.VMEM((2, BLK), jnp.int32),
            pltpu.VMEM((BLK,), jnp.float32),
            pltpu.VMEM((BLK,), jnp.float32),
            pltpu.VMEM((BLK,), jnp.float32),
            pltpu.VMEM((BLK,), jnp.float32),
            pltpu.VMEM((BLK,), jnp.float32),
            pltpu.VMEM((BLK,), jnp.float32),
            pltpu.VMEM((BLK, D), jnp.float32),
            pltpu.VMEM((BLK, D), jnp.float32),
            pltpu.SemaphoreType.DMA,
            pltpu.SemaphoreType.DMA,
            pltpu.SemaphoreType.DMA,
            pltpu.SemaphoreType.DMA,
            pltpu.SemaphoreType.DMA,
            pltpu.SemaphoreType.DMA,
            pltpu.SemaphoreType.DMA,
            pltpu.SemaphoreType.DMA,
            pltpu.VMEM_SHARED((NP,), jnp.float32),
            pltpu.VMEM_SHARED((NP,), jnp.float32),
            pltpu.VMEM_SHARED((NP, D), jnp.float32),
            pltpu.VMEM_SHARED((NP,), jnp.float32),
        ] + ([pltpu.VMEM_SHARED((NP, D), jnp.float32)] if rows_in_spmem else []),
        compiler_params=pltpu.CompilerParams(
            needs_layout_passes=False, use_tc_tiling_on_sc=False),
    )


_dense1 = pl.pallas_call(
    _dense1_body,
    out_shape=[
        jax.ShapeDtypeStruct((NP, HID), jnp.float32),
        jax.ShapeDtypeStruct((NP, 1), jnp.float32),
        jax.ShapeDtypeStruct((NP, 1), jnp.float32),
    ],
)

_combine2 = pl.pallas_call(
    _combine2_body,
    out_shape=[
        jax.ShapeDtypeStruct((NP, CLS), jnp.float32),
        jax.ShapeDtypeStruct((NP, 1), jnp.float32),
        jax.ShapeDtypeStruct((NP, 1), jnp.float32),
    ],
)

_final = pl.pallas_call(
    _final_body,
    out_shape=jax.ShapeDtypeStruct((NP, CLS), jnp.float32),
)


def kernel(features, edges, W1, att_src1, att_dst1, b1, W2, att_src2, att_dst2, b2):
    loop = jnp.arange(N, dtype=edges.dtype)
    src = jnp.concatenate([edges[0], loop, jnp.zeros((EP - E2,), edges.dtype)])
    dst = jnp.concatenate([edges[1], loop, jnp.zeros((EP - E2,), edges.dtype)])
    edges4 = jnp.stack([src.reshape(NW, NB, BLK), dst.reshape(NW, NB, BLK)], axis=2)

    xp = jnp.pad(features, ((0, NP - N), (0, 0)))

    h1, a1s, a1d = _dense1(xp, W1, att_src1, att_dst1)
    part1, s1 = _make_sc_edge(HID)(edges4, a1s.reshape(NP), a1d.reshape(NP), h1)
    h2, a2s, a2d = _combine2(part1, s1, b1, W2, att_src2, att_dst2)
    part2, s2 = _make_sc_edge(CLS, True)(edges4, a2s.reshape(NP), a2d.reshape(NP), h2)
    out = _final(part2, s2, b2)
    return out[:N]


# speedup vs baseline: 42.2157x; 1.3476x over previous
import functools

import jax
import jax.numpy as jnp
from jax import lax
from jax.experimental import pallas as pl
from jax.experimental.pallas import tpu as pltpu, tpu_sc as plsc

N = 10000
NP = 10240
F_IN = 128
HID = 128
CLS = 64
E = 320000
E2 = E + N
NW = 32
BLK = 128
NB = 82
EPT = NB * BLK
EP = NW * EPT
RPS = NP // 16


def _dense1_body(x_ref, w_ref, asrc_ref, adst_ref, h_ref, as_ref, ad_ref):
    h = jnp.dot(x_ref[...], w_ref[...], preferred_element_type=jnp.float32)
    h_ref[...] = h
    as_ref[...] = jnp.sum(h * asrc_ref[...][None, :], axis=1, keepdims=True)
    ad_ref[...] = jnp.sum(h * adst_ref[...][None, :], axis=1, keepdims=True)


def _combine2_body(pa_ref, pb_ref, s_ref, b_ref, w2_ref, asrc_ref, adst_ref,
                   h2_ref, as_ref, ad_ref):
    p = jnp.concatenate([pa_ref[0] + pa_ref[1], pb_ref[0] + pb_ref[1]], axis=1)
    s = s_ref[0] + s_ref[1]
    inv = 1.0 / (s + 1e-16)
    x2 = jnp.maximum(p * jnp.reshape(inv, (NP, 1)) + b_ref[...][None, :], 0.0)
    h2 = jnp.dot(x2, w2_ref[...], preferred_element_type=jnp.float32)
    h2_ref[...] = h2
    as_ref[...] = jnp.sum(h2 * asrc_ref[...][None, :], axis=1, keepdims=True)
    ad_ref[...] = jnp.sum(h2 * adst_ref[...][None, :], axis=1, keepdims=True)


def _final_body(part_ref, s_ref, b_ref, out_ref):
    p = part_ref[0] + part_ref[1]
    s = s_ref[0] + s_ref[1]
    inv = 1.0 / (s + 1e-16)
    z = p * jnp.reshape(inv, (NP, 1)) + b_ref[...][None, :]
    m = jnp.max(z, axis=1, keepdims=True)
    lse = jnp.log(jnp.sum(jnp.exp(z - m), axis=1, keepdims=True)) + m
    out_ref[...] = z - lse


@functools.lru_cache(maxsize=None)
def _make_sc_edge(D, rows_in_spmem=False):
    mesh = plsc.VectorSubcoreMesh(core_axis_name="c", subcore_axis_name="s",
                                  num_cores=2, num_subcores=16)

    def body(edges4, asrc_hbm, adst_hbm, h_hbm, part_out, s_out,
             idxb0, idxb1, asb0, asb1, adb0, adb1, exb0, exb1, rows0, rows1,
             absem0, absem1, grsem0, grsem1, rsem0, rsem1, ssem0, ssem1,
             asrc_sh, adst_sh, acc_sh, s_sh, *maybe_h_sh):
        h_src = maybe_h_sh[0] if rows_in_spmem else h_hbm
        idxb = (idxb0, idxb1)
        asb = (asb0, asb1)
        adb = (adb0, adb1)
        exb = (exb0, exb1)
        rows = (rows0, rows1)
        absem = (absem0, absem1)
        grsem = (grsem0, grsem1)
        rsem = (rsem0, rsem1)
        ssem = (ssem0, ssem1)

        c = lax.axis_index("c")
        s = lax.axis_index("s")
        wid = s * 2 + c
        lane = lax.broadcasted_iota(jnp.int32, (16,), 0)

        def _zb_zero(r, _):
            for v in range(D // 16):
                rows0[r, pl.ds(v * 16, 16)] = jnp.zeros((16,), jnp.float32)
            return 0
        lax.fori_loop(0, BLK, _zb_zero, 0)
        for i in range(BLK // 16):
            exb0[pl.ds(i * 16, 16)] = jnp.zeros((16,), jnp.float32)
        for k in range(RPS // BLK):
            pltpu.sync_copy(rows0, acc_sh.at[pl.ds(s * RPS + k * BLK, BLK)])
            pltpu.sync_copy(exb0, s_sh.at[pl.ds(s * RPS + k * BLK, BLK)])
        @pl.when(s == 0)
        def _():
            pltpu.sync_copy(asrc_hbm, asrc_sh)
            pltpu.sync_copy(adst_hbm, adst_sh)
        if rows_in_spmem:
            pltpu.sync_copy(h_hbm.at[pl.ds(s * RPS, RPS)],
                            maybe_h_sh[0].at[pl.ds(s * RPS, RPS)])
        plsc.subcore_barrier()

        def prefetch(j, b):
            pltpu.sync_copy(edges4.at[wid, j], idxb[b])
            pltpu.async_copy(asrc_sh.at[idxb[b].at[0]], asb[b], absem[b])
            pltpu.async_copy(adst_sh.at[idxb[b].at[1]], adb[b], absem[b])
            pltpu.async_copy(h_src.at[idxb[b].at[0]], rows[b], grsem[b])

        prefetch(0, 0)

        def step(j, b, nb):
            pltpu.make_async_copy(asrc_sh.at[idxb[b].at[0]], asb[b], absem[b]).wait()
            pltpu.make_async_copy(adst_sh.at[idxb[b].at[1]], adb[b], absem[b]).wait()
            base = wid * EPT + j * BLK
            for i in range(BLK // 16):
                t = asb[b][pl.ds(i * 16, 16)] + adb[b][pl.ds(i * 16, 16)]
                e = jnp.where(t > 0, t, 0.2 * t)
                ex = jnp.exp(e)
                gid = base + i * 16 + lane
                ex = jnp.where(gid < E2, ex, 0.0)
                exb[b][pl.ds(i * 16, 16)] = ex
            pltpu.async_copy(exb[b], s_sh.at[idxb[b].at[1]], ssem[b], add=True)

            @pl.when(j + 1 < NB)
            def _():
                @pl.when(j >= 1)
                def _():
                    pltpu.make_async_copy(
                        rows[nb], acc_sh.at[idxb[nb].at[1]], rsem[nb]).wait()
                    pltpu.make_async_copy(
                        exb[nb], s_sh.at[idxb[nb].at[1]], ssem[nb]).wait()
                prefetch(j + 1, nb)

            pltpu.make_async_copy(h_src.at[idxb[b].at[0]], rows[b], grsem[b]).wait()

            @plsc.parallel_loop(0, BLK, step=1, unroll=8)
            def _(i):
                m = plsc.load_gather(exb[b], [jnp.full((16,), i, jnp.int32)])
                for v in range(D // 16):
                    sl = pl.ds(v * 16, 16)
                    rows[b][i, sl] = rows[b][i, sl] * m

            pltpu.async_copy(rows[b], acc_sh.at[idxb[b].at[1]], rsem[b], add=True)

        def outer(g, _):
            step(2 * g, 0, 1)
            step(2 * g + 1, 1, 0)
            return 0
        lax.fori_loop(0, NB // 2, outer, 0)

        for b in (0, 1):
            pltpu.make_async_copy(rows[b], acc_sh.at[idxb[b].at[1]], rsem[b]).wait()
            pltpu.make_async_copy(exb[b], s_sh.at[idxb[b].at[1]], ssem[b]).wait()
        plsc.subcore_barrier()

        off = s * RPS
        pltpu.sync_copy(acc_sh.at[pl.ds(off, RPS)], part_out.at[c, pl.ds(off, RPS)])
        pltpu.sync_copy(s_sh.at[pl.ds(off, RPS)], s_out.at[c, pl.ds(off, RPS)])

    return pl.kernel(
        body,
        out_type=[
            jax.ShapeDtypeStruct((2, NP, D), jnp.float32),
            jax.ShapeDtypeStruct((2, NP), jnp.float32),
        ],
        mesh=mesh,
        scratch_types=[
            pltpu.VMEM((2, BLK), jnp.int32),
            pltpu.VMEM((2, BLK), jnp.int32),
            pltpu.VMEM((BLK,), jnp.float32),
            pltpu.VMEM((BLK,), jnp.float32),
            pltpu.VMEM((BLK,), jnp.float32),
            pltpu.VMEM((BLK,), jnp.float32),
            pltpu.VMEM((BLK,), jnp.float32),
            pltpu.VMEM((BLK,), jnp.float32),
            pltpu.VMEM((BLK, D), jnp.float32),
            pltpu.VMEM((BLK, D), jnp.float32),
            pltpu.SemaphoreType.DMA,
            pltpu.SemaphoreType.DMA,
            pltpu.SemaphoreType.DMA,
            pltpu.SemaphoreType.DMA,
            pltpu.SemaphoreType.DMA,
            pltpu.SemaphoreType.DMA,
            pltpu.SemaphoreType.DMA,
            pltpu.SemaphoreType.DMA,
            pltpu.VMEM_SHARED((NP,), jnp.float32),
            pltpu.VMEM_SHARED((NP,), jnp.float32),
            pltpu.VMEM_SHARED((NP, D), jnp.float32),
            pltpu.VMEM_SHARED((NP,), jnp.float32),
        ] + ([pltpu.VMEM_SHARED((NP, D), jnp.float32)] if rows_in_spmem else []),
        compiler_params=pltpu.CompilerParams(
            needs_layout_passes=False, use_tc_tiling_on_sc=False),
    )


_dense1 = pl.pallas_call(
    _dense1_body,
    out_shape=[
        jax.ShapeDtypeStruct((NP, HID), jnp.float32),
        jax.ShapeDtypeStruct((NP, 1), jnp.float32),
        jax.ShapeDtypeStruct((NP, 1), jnp.float32),
    ],
)

_combine2 = pl.pallas_call(
    _combine2_body,
    out_shape=[
        jax.ShapeDtypeStruct((NP, CLS), jnp.float32),
        jax.ShapeDtypeStruct((NP, 1), jnp.float32),
        jax.ShapeDtypeStruct((NP, 1), jnp.float32),
    ],
)

_final = pl.pallas_call(
    _final_body,
    out_shape=jax.ShapeDtypeStruct((NP, CLS), jnp.float32),
)


def kernel(features, edges, W1, att_src1, att_dst1, b1, W2, att_src2, att_dst2, b2):
    loop = jnp.arange(N, dtype=edges.dtype)
    src = jnp.concatenate([edges[0], loop, jnp.zeros((EP - E2,), edges.dtype)])
    dst = jnp.concatenate([edges[1], loop, jnp.zeros((EP - E2,), edges.dtype)])
    edges4 = jnp.stack([src.reshape(NW, NB, BLK), dst.reshape(NW, NB, BLK)], axis=2)

    xp = jnp.pad(features, ((0, NP - N), (0, 0)))

    h1, a1s, a1d = _dense1(xp, W1, att_src1, att_dst1)
    sc = _make_sc_edge(CLS, True)
    pa, s1 = sc(edges4, a1s.reshape(NP), a1d.reshape(NP), h1[:, :CLS])
    pb, _ = sc(edges4, a1s.reshape(NP), a1d.reshape(NP), h1[:, CLS:])
    h2, a2s, a2d = _combine2(pa, pb, s1, b1, W2, att_src2, att_dst2)
    part2, s2 = sc(edges4, a2s.reshape(NP), a2d.reshape(NP), h2)
    out = _final(part2, s2, b2)
    return out[:N]


# speedup vs baseline: 46.8272x; 1.1092x over previous
import functools

import jax
import jax.numpy as jnp
from jax import lax
from jax.experimental import pallas as pl
from jax.experimental.pallas import tpu as pltpu, tpu_sc as plsc

N = 10000
NP = 10240
F_IN = 128
HID = 128
CLS = 64
E = 320000
E2 = E + N
NW = 32
BLK = 128
NB = 84
EPT = NB * BLK
EP = NW * EPT
RPS = NP // 16
DEPTH = 4


def _dense1_body(x_ref, w_ref, asrc_ref, adst_ref, h_ref, as_ref, ad_ref):
    h = jnp.dot(x_ref[...], w_ref[...], preferred_element_type=jnp.float32)
    h_ref[...] = h
    as_ref[...] = jnp.sum(h * asrc_ref[...][None, :], axis=1, keepdims=True)
    ad_ref[...] = jnp.sum(h * adst_ref[...][None, :], axis=1, keepdims=True)


def _combine2_body(pa_ref, pb_ref, s_ref, b_ref, w2_ref, asrc_ref, adst_ref,
                   h2_ref, as_ref, ad_ref):
    p = jnp.concatenate([pa_ref[0] + pa_ref[1], pb_ref[0] + pb_ref[1]], axis=1)
    s = s_ref[0] + s_ref[1]
    inv = 1.0 / (s + 1e-16)
    x2 = jnp.maximum(p * jnp.reshape(inv, (NP, 1)) + b_ref[...][None, :], 0.0)
    h2 = jnp.dot(x2, w2_ref[...], preferred_element_type=jnp.float32)
    h2_ref[...] = h2
    as_ref[...] = jnp.sum(h2 * asrc_ref[...][None, :], axis=1, keepdims=True)
    ad_ref[...] = jnp.sum(h2 * adst_ref[...][None, :], axis=1, keepdims=True)


def _final_body(part_ref, s_ref, b_ref, out_ref):
    p = part_ref[0] + part_ref[1]
    s = s_ref[0] + s_ref[1]
    inv = 1.0 / (s + 1e-16)
    z = p * jnp.reshape(inv, (NP, 1)) + b_ref[...][None, :]
    m = jnp.max(z, axis=1, keepdims=True)
    lse = jnp.log(jnp.sum(jnp.exp(z - m), axis=1, keepdims=True)) + m
    out_ref[...] = z - lse


@functools.lru_cache(maxsize=None)
def _make_sc_edge(D):
    mesh = plsc.VectorSubcoreMesh(core_axis_name="c", subcore_axis_name="s",
                                  num_cores=2, num_subcores=16)

    def body(edges4, asrc_hbm, adst_hbm, h_hbm, part_out, s_out,
             idxb, asb, adb, exb, rows, isem, absem, grsem, rsem, ssem,
             asrc_sh, adst_sh, acc_sh, s_sh, h_sh):
        c = lax.axis_index("c")
        s = lax.axis_index("s")
        wid = s * 2 + c
        lane = lax.broadcasted_iota(jnp.int32, (16,), 0)

        def _zb_zero(r, _):
            for v in range(D // 16):
                rows[0][r, pl.ds(v * 16, 16)] = jnp.zeros((16,), jnp.float32)
            return 0
        lax.fori_loop(0, BLK, _zb_zero, 0)
        for i in range(BLK // 16):
            exb[0][pl.ds(i * 16, 16)] = jnp.zeros((16,), jnp.float32)
        for k in range(RPS // BLK):
            pltpu.sync_copy(rows[0], acc_sh.at[pl.ds(s * RPS + k * BLK, BLK)])
            pltpu.sync_copy(exb[0], s_sh.at[pl.ds(s * RPS + k * BLK, BLK)])
        @pl.when(s == 0)
        def _():
            pltpu.sync_copy(asrc_hbm, asrc_sh)
            pltpu.sync_copy(adst_hbm, adst_sh)
        pltpu.sync_copy(h_hbm.at[pl.ds(s * RPS, RPS)],
                        h_sh.at[pl.ds(s * RPS, RPS)])
        plsc.subcore_barrier()

        def idx_load(j, k):
            pltpu.async_copy(edges4.at[wid, j], idxb[k], isem[k])

        def gathers(j, k):
            pltpu.make_async_copy(edges4.at[wid, j], idxb[k], isem[k]).wait()
            pltpu.async_copy(asrc_sh.at[idxb[k].at[0]], asb[k], absem[k])
            pltpu.async_copy(adst_sh.at[idxb[k].at[1]], adb[k], absem[k])
            pltpu.async_copy(h_sh.at[idxb[k].at[0]], rows[k], grsem[k])

        def drain_scatters(k):
            pltpu.make_async_copy(rows[k], acc_sh.at[idxb[k].at[1]], rsem[k]).wait()
            pltpu.make_async_copy(exb[k], s_sh.at[idxb[k].at[1]], ssem[k]).wait()

        idx_load(0, 0)
        idx_load(1, 1)
        gathers(0, 0)

        def step(j, k, k1, k2):
            @pl.when(j >= 2)
            def _():
                drain_scatters(k2)
            @pl.when(j + 2 < NB)
            def _():
                idx_load(j + 2, k2)
            @pl.when(j + 1 < NB)
            def _():
                gathers(j + 1, k1)

            pltpu.make_async_copy(asrc_sh.at[idxb[k].at[0]], asb[k], absem[k]).wait()
            pltpu.make_async_copy(adst_sh.at[idxb[k].at[1]], adb[k], absem[k]).wait()
            base = wid * EPT + j * BLK
            for i in range(BLK // 16):
                t = asb[k][pl.ds(i * 16, 16)] + adb[k][pl.ds(i * 16, 16)]
                e = jnp.where(t > 0, t, 0.2 * t)
                ex = jnp.exp(e)
                gid = base + i * 16 + lane
                ex = jnp.where(gid < E2, ex, 0.0)
                exb[k][pl.ds(i * 16, 16)] = ex
            pltpu.async_copy(exb[k], s_sh.at[idxb[k].at[1]], ssem[k], add=True)

            pltpu.make_async_copy(h_sh.at[idxb[k].at[0]], rows[k], grsem[k]).wait()

            @plsc.parallel_loop(0, BLK, step=1, unroll=8)
            def _(i):
                m = plsc.load_gather(exb[k], [jnp.full((16,), i, jnp.int32)])
                for v in range(D // 16):
                    sl = pl.ds(v * 16, 16)
                    rows[k][i, sl] = rows[k][i, sl] * m

            pltpu.async_copy(rows[k], acc_sh.at[idxb[k].at[1]], rsem[k], add=True)

        def outer(g, _):
            for k in range(DEPTH):
                j = DEPTH * g + k
                step(j, k, (k + 1) % DEPTH, (k + 2) % DEPTH)
            return 0
        lax.fori_loop(0, NB // DEPTH, outer, 0)

        drain_scatters((NB - 2) % DEPTH)
        drain_scatters((NB - 1) % DEPTH)
        plsc.subcore_barrier()

        off = s * RPS
        pltpu.sync_copy(acc_sh.at[pl.ds(off, RPS)], part_out.at[c, pl.ds(off, RPS)])
        pltpu.sync_copy(s_sh.at[pl.ds(off, RPS)], s_out.at[c, pl.ds(off, RPS)])

    return pl.kernel(
        body,
        out_type=[
            jax.ShapeDtypeStruct((2, NP, D), jnp.float32),
            jax.ShapeDtypeStruct((2, NP), jnp.float32),
        ],
        mesh=mesh,
        scratch_types=[
            [pltpu.VMEM((2, BLK), jnp.int32) for _ in range(DEPTH)],
            [pltpu.VMEM((BLK,), jnp.float32) for _ in range(DEPTH)],
            [pltpu.VMEM((BLK,), jnp.float32) for _ in range(DEPTH)],
            [pltpu.VMEM((BLK,), jnp.float32) for _ in range(DEPTH)],
            [pltpu.VMEM((BLK, D), jnp.float32) for _ in range(DEPTH)],
            [pltpu.SemaphoreType.DMA for _ in range(DEPTH)],
            [pltpu.SemaphoreType.DMA for _ in range(DEPTH)],
            [pltpu.SemaphoreType.DMA for _ in range(DEPTH)],
            [pltpu.SemaphoreType.DMA for _ in range(DEPTH)],
            [pltpu.SemaphoreType.DMA for _ in range(DEPTH)],
            pltpu.VMEM_SHARED((NP,), jnp.float32),
            pltpu.VMEM_SHARED((NP,), jnp.float32),
            pltpu.VMEM_SHARED((NP, D), jnp.float32),
            pltpu.VMEM_SHARED((NP,), jnp.float32),
            pltpu.VMEM_SHARED((NP, D), jnp.float32),
        ],
        compiler_params=pltpu.CompilerParams(
            needs_layout_passes=False, use_tc_tiling_on_sc=False),
    )


_dense1 = pl.pallas_call(
    _dense1_body,
    out_shape=[
        jax.ShapeDtypeStruct((NP, HID), jnp.float32),
        jax.ShapeDtypeStruct((NP, 1), jnp.float32),
        jax.ShapeDtypeStruct((NP, 1), jnp.float32),
    ],
)

_combine2 = pl.pallas_call(
    _combine2_body,
    out_shape=[
        jax.ShapeDtypeStruct((NP, CLS), jnp.float32),
        jax.ShapeDtypeStruct((NP, 1), jnp.float32),
        jax.ShapeDtypeStruct((NP, 1), jnp.float32),
    ],
)

_final = pl.pallas_call(
    _final_body,
    out_shape=jax.ShapeDtypeStruct((NP, CLS), jnp.float32),
)


def kernel(features, edges, W1, att_src1, att_dst1, b1, W2, att_src2, att_dst2, b2):
    loop = jnp.arange(N, dtype=edges.dtype)
    src = jnp.concatenate([edges[0], loop, jnp.zeros((EP - E2,), edges.dtype)])
    dst = jnp.concatenate([edges[1], loop, jnp.zeros((EP - E2,), edges.dtype)])
    edges4 = jnp.stack([src.reshape(NW, NB, BLK), dst.reshape(NW, NB, BLK)], axis=2)

    xp = jnp.pad(features, ((0, NP - N), (0, 0)))

    h1, a1s, a1d = _dense1(xp, W1, att_src1, att_dst1)
    sc = _make_sc_edge(CLS)
    pa, s1 = sc(edges4, a1s.reshape(NP), a1d.reshape(NP), h1[:, :CLS])
    pb, _ = sc(edges4, a1s.reshape(NP), a1d.reshape(NP), h1[:, CLS:])
    h2, a2s, a2d = _combine2(pa, pb, s1, b1, W2, att_src2, att_dst2)
    part2, s2 = sc(edges4, a2s.reshape(NP), a2d.reshape(NP), h2)
    out = _final(part2, s2, b2)
    return out[:N]


# speedup vs baseline: 54.6496x; 1.1670x over previous
import functools

import jax
import jax.numpy as jnp
from jax import lax
from jax.experimental import pallas as pl
from jax.experimental.pallas import tpu as pltpu, tpu_sc as plsc

N = 10000
NP = 10240
F_IN = 128
HID = 128
CLS = 64
E = 320000
NW = 32
BLK = 128
NB = 80
EPT = NB * BLK
EP = NW * EPT
RPS = NP // 16
DEPTH = 4


def _self_ex(as_col, ad_col):
    t = as_col + ad_col
    return jnp.exp(jnp.where(t > 0, t, 0.2 * t))


def _dense1_body(x_ref, w_ref, asrc_ref, adst_ref, ha_ref, hb_ref, as_ref, ad_ref):
    h = jnp.dot(x_ref[...], w_ref[...], preferred_element_type=jnp.float32)
    ha_ref[...] = h[:, :CLS]
    hb_ref[...] = h[:, CLS:]
    as_ref[...] = jnp.sum(h * asrc_ref[...][None, :], axis=1)
    ad_ref[...] = jnp.sum(h * adst_ref[...][None, :], axis=1)


def _combine2_body(pa_ref, pb_ref, s_ref, ha_ref, hb_ref, a1s_ref, a1d_ref,
                   b_ref, w2_ref, asrc_ref, adst_ref, h2_ref, as_ref, ad_ref):
    ex0 = jnp.reshape(_self_ex(a1s_ref[...], a1d_ref[...]), (NP, 1))
    p = jnp.concatenate([pa_ref[0] + pa_ref[1] + ex0 * ha_ref[...],
                         pb_ref[0] + pb_ref[1] + ex0 * hb_ref[...]], axis=1)
    s = jnp.reshape(s_ref[0] + s_ref[1], (NP, 1)) + ex0
    inv = 1.0 / (s + 1e-16)
    x2 = jnp.maximum(p * inv + b_ref[...][None, :], 0.0)
    h2 = jnp.dot(x2, w2_ref[...], preferred_element_type=jnp.float32)
    h2_ref[...] = h2
    as_ref[...] = jnp.sum(h2 * asrc_ref[...][None, :], axis=1)
    ad_ref[...] = jnp.sum(h2 * adst_ref[...][None, :], axis=1)


def _final_body(part_ref, s_ref, h2_ref, a2s_ref, a2d_ref, b_ref, out_ref):
    ex0 = jnp.reshape(_self_ex(a2s_ref[...], a2d_ref[...]), (NP, 1))
    p = part_ref[0] + part_ref[1] + ex0 * h2_ref[...]
    s = jnp.reshape(s_ref[0] + s_ref[1], (NP, 1)) + ex0
    inv = 1.0 / (s + 1e-16)
    z = p * inv + b_ref[...][None, :]
    m = jnp.max(z, axis=1, keepdims=True)
    lse = jnp.log(jnp.sum(jnp.exp(z - m), axis=1, keepdims=True)) + m
    out_ref[...] = z - lse


@functools.lru_cache(maxsize=None)
def _make_sc_edge(D):
    mesh = plsc.VectorSubcoreMesh(core_axis_name="c", subcore_axis_name="s",
                                  num_cores=2, num_subcores=16)

    def body(edges_p, asrc_hbm, adst_hbm, h_hbm, part_out, s_out,
             idxb, asb, adb, exb, rows, isem, absem, grsem, rsem, ssem,
             asrc_sh, adst_sh, acc_sh, s_sh, h_sh):
        c = lax.axis_index("c")
        s = lax.axis_index("s")
        wid = s * 2 + c
        lane = lax.broadcasted_iota(jnp.int32, (16,), 0)

        def _zb_zero(r, _):
            for v in range(D // 16):
                rows[0][r, pl.ds(v * 16, 16)] = jnp.zeros((16,), jnp.float32)
            return 0
        lax.fori_loop(0, BLK, _zb_zero, 0)
        for i in range(BLK // 16):
            exb[0][pl.ds(i * 16, 16)] = jnp.zeros((16,), jnp.float32)
        for k in range(RPS // BLK):
            pltpu.sync_copy(rows[0], acc_sh.at[pl.ds(s * RPS + k * BLK, BLK)])
            pltpu.sync_copy(exb[0], s_sh.at[pl.ds(s * RPS + k * BLK, BLK)])
        @pl.when(s == 0)
        def _():
            pltpu.sync_copy(asrc_hbm, asrc_sh)
            pltpu.sync_copy(adst_hbm, adst_sh)
        pltpu.sync_copy(h_hbm.at[pl.ds(s * RPS, RPS)],
                        h_sh.at[pl.ds(s * RPS, RPS)])
        plsc.subcore_barrier()

        def idx_load(j, k):
            base = wid * EPT + j * BLK
            pltpu.async_copy(edges_p.at[0, pl.ds(base, BLK)], idxb[k].at[0], isem[k])
            pltpu.async_copy(edges_p.at[1, pl.ds(base, BLK)], idxb[k].at[1], isem[k])

        def gathers(j, k):
            base = wid * EPT + j * BLK
            pltpu.make_async_copy(edges_p.at[0, pl.ds(base, BLK)], idxb[k].at[0], isem[k]).wait()
            pltpu.make_async_copy(edges_p.at[1, pl.ds(base, BLK)], idxb[k].at[1], isem[k]).wait()
            pltpu.async_copy(asrc_sh.at[idxb[k].at[0]], asb[k], absem[k])
            pltpu.async_copy(adst_sh.at[idxb[k].at[1]], adb[k], absem[k])
            pltpu.async_copy(h_sh.at[idxb[k].at[0]], rows[k], grsem[k])

        def drain_scatters(k):
            pltpu.make_async_copy(rows[k], acc_sh.at[idxb[k].at[1]], rsem[k]).wait()
            pltpu.make_async_copy(exb[k], s_sh.at[idxb[k].at[1]], ssem[k]).wait()

        idx_load(0, 0)
        idx_load(1, 1)
        gathers(0, 0)

        def step(j, k, k1, k2):
            @pl.when(j >= 2)
            def _():
                drain_scatters(k2)
            @pl.when(j + 2 < NB)
            def _():
                idx_load(j + 2, k2)
            @pl.when(j + 1 < NB)
            def _():
                gathers(j + 1, k1)

            pltpu.make_async_copy(asrc_sh.at[idxb[k].at[0]], asb[k], absem[k]).wait()
            pltpu.make_async_copy(adst_sh.at[idxb[k].at[1]], adb[k], absem[k]).wait()
            base = wid * EPT + j * BLK
            for i in range(BLK // 16):
                t = asb[k][pl.ds(i * 16, 16)] + adb[k][pl.ds(i * 16, 16)]
                e = jnp.where(t > 0, t, 0.2 * t)
                ex = jnp.exp(e)
                gid = base + i * 16 + lane
                ex = jnp.where(gid < E, ex, 0.0)
                exb[k][pl.ds(i * 16, 16)] = ex
            pltpu.async_copy(exb[k], s_sh.at[idxb[k].at[1]], ssem[k], add=True)

            pltpu.make_async_copy(h_sh.at[idxb[k].at[0]], rows[k], grsem[k]).wait()

            @plsc.parallel_loop(0, BLK, step=1, unroll=8)
            def _(i):
                m = plsc.load_gather(exb[k], [jnp.full((16,), i, jnp.int32)])
                for v in range(D // 16):
                    sl = pl.ds(v * 16, 16)
                    rows[k][i, sl] = rows[k][i, sl] * m

            pltpu.async_copy(rows[k], acc_sh.at[idxb[k].at[1]], rsem[k], add=True)

        def outer(g, _):
            for k in range(DEPTH):
                j = DEPTH * g + k
                step(j, k, (k + 1) % DEPTH, (k + 2) % DEPTH)
            return 0
        lax.fori_loop(0, NB // DEPTH, outer, 0)

        drain_scatters((NB - 2) % DEPTH)
        drain_scatters((NB - 1) % DEPTH)
        plsc.subcore_barrier()

        off = s * RPS
        pltpu.sync_copy(acc_sh.at[pl.ds(off, RPS)], part_out.at[c, pl.ds(off, RPS)])
        pltpu.sync_copy(s_sh.at[pl.ds(off, RPS)], s_out.at[c, pl.ds(off, RPS)])

    return pl.kernel(
        body,
        out_type=[
            jax.ShapeDtypeStruct((2, NP, D), jnp.float32),
            jax.ShapeDtypeStruct((2, NP), jnp.float32),
        ],
        mesh=mesh,
        scratch_types=[
            [pltpu.VMEM((2, BLK), jnp.int32) for _ in range(DEPTH)],
            [pltpu.VMEM((BLK,), jnp.float32) for _ in range(DEPTH)],
            [pltpu.VMEM((BLK,), jnp.float32) for _ in range(DEPTH)],
            [pltpu.VMEM((BLK,), jnp.float32) for _ in range(DEPTH)],
            [pltpu.VMEM((BLK, D), jnp.float32) for _ in range(DEPTH)],
            [pltpu.SemaphoreType.DMA for _ in range(DEPTH)],
            [pltpu.SemaphoreType.DMA for _ in range(DEPTH)],
            [pltpu.SemaphoreType.DMA for _ in range(DEPTH)],
            [pltpu.SemaphoreType.DMA for _ in range(DEPTH)],
            [pltpu.SemaphoreType.DMA for _ in range(DEPTH)],
            pltpu.VMEM_SHARED((NP,), jnp.float32),
            pltpu.VMEM_SHARED((NP,), jnp.float32),
            pltpu.VMEM_SHARED((NP, D), jnp.float32),
            pltpu.VMEM_SHARED((NP,), jnp.float32),
            pltpu.VMEM_SHARED((NP, D), jnp.float32),
        ],
        compiler_params=pltpu.CompilerParams(
            needs_layout_passes=False, use_tc_tiling_on_sc=False),
    )


_dense1 = pl.pallas_call(
    _dense1_body,
    out_shape=[
        jax.ShapeDtypeStruct((NP, CLS), jnp.float32),
        jax.ShapeDtypeStruct((NP, CLS), jnp.float32),
        jax.ShapeDtypeStruct((NP,), jnp.float32),
        jax.ShapeDtypeStruct((NP,), jnp.float32),
    ],
)

_combine2 = pl.pallas_call(
    _combine2_body,
    out_shape=[
        jax.ShapeDtypeStruct((NP, CLS), jnp.float32),
        jax.ShapeDtypeStruct((NP,), jnp.float32),
        jax.ShapeDtypeStruct((NP,), jnp.float32),
    ],
)

_final = pl.pallas_call(
    _final_body,
    out_shape=jax.ShapeDtypeStruct((NP, CLS), jnp.float32),
)


def kernel(features, edges, W1, att_src1, att_dst1, b1, W2, att_src2, att_dst2, b2):
    edges_p = jnp.pad(edges, ((0, 0), (0, EP - E)))
    xp = jnp.pad(features, ((0, NP - N), (0, 0)))

    h1a, h1b, a1s, a1d = _dense1(xp, W1, att_src1, att_dst1)
    sc = _make_sc_edge(CLS)
    pa, s1 = sc(edges_p, a1s, a1d, h1a)
    pb, _ = sc(edges_p, a1s, a1d, h1b)
    h2, a2s, a2d = _combine2(pa, pb, s1, h1a, h1b, a1s, a1d,
                             b1, W2, att_src2, att_dst2)
    part2, s2 = sc(edges_p, a2s, a2d, h2)
    out = _final(part2, s2, h2, a2s, a2d, b2)
    return out[:N]


# speedup vs baseline: 58.5662x; 1.0717x over previous
import functools

import jax
import jax.numpy as jnp
from jax import lax
from jax.experimental import pallas as pl
from jax.experimental.pallas import tpu as pltpu, tpu_sc as plsc

N = 10000
NP = 10240
F_IN = 128
HID = 128
CLS = 64
E = 320000
NW = 32
BLK = 128
NB = 80
EPT = NB * BLK
EP = NW * EPT
RPS = NP // 16
DEPTH = 4


def _self_ex(as_col, ad_col):
    t = as_col + ad_col
    return jnp.exp(jnp.where(t > 0, t, 0.2 * t))


def _dense1_body(x_ref, w_ref, asrc_ref, adst_ref, h_ref, as_ref, ad_ref):
    h = jnp.dot(x_ref[...], w_ref[...], preferred_element_type=jnp.float32)
    h_ref[...] = h
    as_ref[...] = jnp.sum(h * asrc_ref[...][None, :], axis=1)
    ad_ref[...] = jnp.sum(h * adst_ref[...][None, :], axis=1)


def _combine2_body(pa_ref, pb_ref, s_ref, h1_ref, a1s_ref, a1d_ref,
                   b_ref, w2_ref, asrc_ref, adst_ref, h2_ref, as_ref, ad_ref):
    ex0 = jnp.reshape(_self_ex(a1s_ref[...], a1d_ref[...]), (NP, 1))
    pa = pa_ref[...]
    pb = pb_ref[...]
    p = (jnp.concatenate([pa[:, :CLS] + pa[:, CLS:], pb[:, :CLS] + pb[:, CLS:]],
                         axis=1) + ex0 * h1_ref[...])
    s = jnp.reshape(s_ref[0] + s_ref[1], (NP, 1)) + ex0
    inv = 1.0 / (s + 1e-16)
    x2 = jnp.maximum(p * inv + b_ref[...][None, :], 0.0)
    h2 = jnp.dot(x2, w2_ref[...], preferred_element_type=jnp.float32)
    h2_ref[...] = jnp.concatenate([h2, h2], axis=1)
    as_ref[...] = jnp.sum(h2 * asrc_ref[...][None, :], axis=1)
    ad_ref[...] = jnp.sum(h2 * adst_ref[...][None, :], axis=1)


def _final_body(part_ref, s_ref, h2_ref, a2s_ref, a2d_ref, b_ref, out_ref):
    ex0 = jnp.reshape(_self_ex(a2s_ref[...], a2d_ref[...]), (NP, 1))
    pt = part_ref[...]
    p = pt[:, :CLS] + pt[:, CLS:] + ex0 * h2_ref[:, :CLS]
    s = jnp.reshape(s_ref[0] + s_ref[1], (NP, 1)) + ex0
    inv = 1.0 / (s + 1e-16)
    z = p * inv + b_ref[...][None, :]
    m = jnp.max(z, axis=1, keepdims=True)
    lse = jnp.log(jnp.sum(jnp.exp(z - m), axis=1, keepdims=True)) + m
    out_ref[...] = z - lse


@functools.lru_cache(maxsize=None)
def _make_sc_edge(hoff):
    D = CLS
    mesh = plsc.VectorSubcoreMesh(core_axis_name="c", subcore_axis_name="s",
                                  num_cores=2, num_subcores=16)

    def body(edges_p, asrc_hbm, adst_hbm, h_hbm, part_out, s_out,
             idxb, asb, adb, exb, rows, isem, absem, grsem, rsem, ssem,
             asrc_sh, adst_sh, acc_sh, s_sh, h_sh):
        c = lax.axis_index("c")
        s = lax.axis_index("s")
        wid = s * 2 + c
        lane = lax.broadcasted_iota(jnp.int32, (16,), 0)

        def _zb_zero(r, _):
            for v in range(D // 16):
                rows[0][r, pl.ds(v * 16, 16)] = jnp.zeros((16,), jnp.float32)
            return 0
        lax.fori_loop(0, BLK, _zb_zero, 0)
        for i in range(BLK // 16):
            exb[0][pl.ds(i * 16, 16)] = jnp.zeros((16,), jnp.float32)
        for k in range(RPS // BLK):
            pltpu.sync_copy(rows[0], acc_sh.at[pl.ds(s * RPS + k * BLK, BLK)])
            pltpu.sync_copy(exb[0], s_sh.at[pl.ds(s * RPS + k * BLK, BLK)])
        @pl.when(s == 0)
        def _():
            pltpu.sync_copy(asrc_hbm, asrc_sh)
            pltpu.sync_copy(adst_hbm, adst_sh)
        pltpu.sync_copy(h_hbm.at[pl.ds(s * RPS, RPS), pl.ds(hoff, CLS)],
                        h_sh.at[pl.ds(s * RPS, RPS)])
        plsc.subcore_barrier()

        def idx_load(j, k):
            base = wid * EPT + j * BLK
            pltpu.async_copy(edges_p.at[0, pl.ds(base, BLK)], idxb[k].at[0], isem[k])
            pltpu.async_copy(edges_p.at[1, pl.ds(base, BLK)], idxb[k].at[1], isem[k])

        def gathers(j, k):
            base = wid * EPT + j * BLK
            pltpu.make_async_copy(edges_p.at[0, pl.ds(base, BLK)], idxb[k].at[0], isem[k]).wait()
            pltpu.make_async_copy(edges_p.at[1, pl.ds(base, BLK)], idxb[k].at[1], isem[k]).wait()
            pltpu.async_copy(asrc_sh.at[idxb[k].at[0]], asb[k], absem[k])
            pltpu.async_copy(adst_sh.at[idxb[k].at[1]], adb[k], absem[k])
            pltpu.async_copy(h_sh.at[idxb[k].at[0]], rows[k], grsem[k])

        def drain_scatters(k):
            pltpu.make_async_copy(rows[k], acc_sh.at[idxb[k].at[1]], rsem[k]).wait()
            pltpu.make_async_copy(exb[k], s_sh.at[idxb[k].at[1]], ssem[k]).wait()

        idx_load(0, 0)
        idx_load(1, 1)
        gathers(0, 0)

        def step(j, k, k1, k2):
            @pl.when(j >= 2)
            def _():
                drain_scatters(k2)
            @pl.when(j + 2 < NB)
            def _():
                idx_load(j + 2, k2)
            @pl.when(j + 1 < NB)
            def _():
                gathers(j + 1, k1)

            pltpu.make_async_copy(asrc_sh.at[idxb[k].at[0]], asb[k], absem[k]).wait()
            pltpu.make_async_copy(adst_sh.at[idxb[k].at[1]], adb[k], absem[k]).wait()
            base = wid * EPT + j * BLK
            for i in range(BLK // 16):
                t = asb[k][pl.ds(i * 16, 16)] + adb[k][pl.ds(i * 16, 16)]
                e = jnp.where(t > 0, t, 0.2 * t)
                ex = jnp.exp(e)
                gid = base + i * 16 + lane
                ex = jnp.where(gid < E, ex, 0.0)
                exb[k][pl.ds(i * 16, 16)] = ex
            pltpu.async_copy(exb[k], s_sh.at[idxb[k].at[1]], ssem[k], add=True)

            pltpu.make_async_copy(h_sh.at[idxb[k].at[0]], rows[k], grsem[k]).wait()

            @plsc.parallel_loop(0, BLK, step=1, unroll=8)
            def _(i):
                m = plsc.load_gather(exb[k], [jnp.full((16,), i, jnp.int32)])
                for v in range(D // 16):
                    sl = pl.ds(v * 16, 16)
                    rows[k][i, sl] = rows[k][i, sl] * m

            pltpu.async_copy(rows[k], acc_sh.at[idxb[k].at[1]], rsem[k], add=True)

        def outer(g, _):
            for k in range(DEPTH):
                j = DEPTH * g + k
                step(j, k, (k + 1) % DEPTH, (k + 2) % DEPTH)
            return 0
        lax.fori_loop(0, NB // DEPTH, outer, 0)

        drain_scatters((NB - 2) % DEPTH)
        drain_scatters((NB - 1) % DEPTH)
        plsc.subcore_barrier()

        off = s * RPS
        @pl.when(c == 0)
        def _():
            pltpu.sync_copy(acc_sh.at[pl.ds(off, RPS)],
                            part_out.at[pl.ds(off, RPS), pl.ds(0, CLS)])
        @pl.when(c == 1)
        def _():
            pltpu.sync_copy(acc_sh.at[pl.ds(off, RPS)],
                            part_out.at[pl.ds(off, RPS), pl.ds(CLS, CLS)])
        pltpu.sync_copy(s_sh.at[pl.ds(off, RPS)], s_out.at[c, pl.ds(off, RPS)])

    return pl.kernel(
        body,
        out_type=[
            jax.ShapeDtypeStruct((NP, HID), jnp.float32),
            jax.ShapeDtypeStruct((2, NP), jnp.float32),
        ],
        mesh=mesh,
        scratch_types=[
            [pltpu.VMEM((2, BLK), jnp.int32) for _ in range(DEPTH)],
            [pltpu.VMEM((BLK,), jnp.float32) for _ in range(DEPTH)],
            [pltpu.VMEM((BLK,), jnp.float32) for _ in range(DEPTH)],
            [pltpu.VMEM((BLK,), jnp.float32) for _ in range(DEPTH)],
            [pltpu.VMEM((BLK, D), jnp.float32) for _ in range(DEPTH)],
            [pltpu.SemaphoreType.DMA for _ in range(DEPTH)],
            [pltpu.SemaphoreType.DMA for _ in range(DEPTH)],
            [pltpu.SemaphoreType.DMA for _ in range(DEPTH)],
            [pltpu.SemaphoreType.DMA for _ in range(DEPTH)],
            [pltpu.SemaphoreType.DMA for _ in range(DEPTH)],
            pltpu.VMEM_SHARED((NP,), jnp.float32),
            pltpu.VMEM_SHARED((NP,), jnp.float32),
            pltpu.VMEM_SHARED((NP, D), jnp.float32),
            pltpu.VMEM_SHARED((NP,), jnp.float32),
            pltpu.VMEM_SHARED((NP, D), jnp.float32),
        ],
        compiler_params=pltpu.CompilerParams(
            needs_layout_passes=False, use_tc_tiling_on_sc=False),
    )


_dense1 = pl.pallas_call(
    _dense1_body,
    out_shape=[
        jax.ShapeDtypeStruct((NP, HID), jnp.float32),
        jax.ShapeDtypeStruct((NP,), jnp.float32),
        jax.ShapeDtypeStruct((NP,), jnp.float32),
    ],
)

_combine2 = pl.pallas_call(
    _combine2_body,
    out_shape=[
        jax.ShapeDtypeStruct((NP, HID), jnp.float32),
        jax.ShapeDtypeStruct((NP,), jnp.float32),
        jax.ShapeDtypeStruct((NP,), jnp.float32),
    ],
)

_final = pl.pallas_call(
    _final_body,
    out_shape=jax.ShapeDtypeStruct((NP, CLS), jnp.float32),
)


def kernel(features, edges, W1, att_src1, att_dst1, b1, W2, att_src2, att_dst2, b2):
    edges_p = jnp.pad(edges, ((0, 0), (0, EP - E)))
    xp = jnp.pad(features, ((0, NP - N), (0, 0)))

    h1, a1s, a1d = _dense1(xp, W1, att_src1, att_dst1)
    pa, s1 = _make_sc_edge(0)(edges_p, a1s, a1d, h1)
    pb, _ = _make_sc_edge(CLS)(edges_p, a1s, a1d, h1)
    h2, a2s, a2d = _combine2(pa, pb, s1, h1, a1s, a1d,
                             b1, W2, att_src2, att_dst2)
    part2, s2 = _make_sc_edge(0)(edges_p, a2s, a2d, h2)
    out = _final(part2, s2, h2, a2s, a2d, b2)
    return out[:N]


# speedup vs baseline: 62.4467x; 1.0663x over previous
import functools

import jax
import jax.numpy as jnp
from jax import lax
from jax.experimental import pallas as pl
from jax.experimental.pallas import tpu as pltpu, tpu_sc as plsc

N = 10000
NP = 10240
F_IN = 128
HID = 128
CLS = 64
E = 320000
NW = 32
BLK = 128
NB = 80
EPT = NB * BLK
EP = NW * EPT
RPS = NP // 16
DEPTH = 4


def _self_ex(as_col, ad_col):
    t = as_col + ad_col
    return jnp.exp(jnp.where(t > 0, t, 0.2 * t))


def _dense1_body(x_ref, w_ref, asrc_ref, adst_ref, h_ref, as_ref, ad_ref):
    h = jnp.dot(x_ref[...], w_ref[...], preferred_element_type=jnp.float32)
    h_ref[...] = h
    as_ref[...] = jnp.sum(h * asrc_ref[...][None, :], axis=1)
    ad_ref[...] = jnp.sum(h * adst_ref[...][None, :], axis=1)


def _combine2_body(pa_ref, pb_ref, s_ref, h1_ref, a1s_ref, a1d_ref,
                   b_ref, w2_ref, asrc_ref, adst_ref, h2_ref, as_ref, ad_ref):
    ex0 = jnp.reshape(_self_ex(a1s_ref[...], a1d_ref[...]), (NP, 1))
    pa = pa_ref[...]
    pb = pb_ref[...]
    p = (jnp.concatenate([pa[:, :CLS] + pa[:, CLS:], pb[:, :CLS] + pb[:, CLS:]],
                         axis=1) + ex0 * h1_ref[...])
    s = jnp.reshape(s_ref[0] + s_ref[1], (NP, 1)) + ex0
    inv = 1.0 / (s + 1e-16)
    x2 = jnp.maximum(p * inv + b_ref[...][None, :], 0.0)
    h2 = jnp.dot(x2, w2_ref[...], preferred_element_type=jnp.float32)
    h2_ref[...] = jnp.concatenate([h2, h2], axis=1)
    as_ref[...] = jnp.sum(h2 * asrc_ref[...][None, :], axis=1)
    ad_ref[...] = jnp.sum(h2 * adst_ref[...][None, :], axis=1)


def _final_body(part_ref, s_ref, h2_ref, a2s_ref, a2d_ref, b_ref, out_ref):
    ex0 = jnp.reshape(_self_ex(a2s_ref[...], a2d_ref[...]), (NP, 1))
    pt = part_ref[...]
    p = pt[:, :CLS] + pt[:, CLS:] + ex0 * h2_ref[:, :CLS]
    s = jnp.reshape(s_ref[0] + s_ref[1], (NP, 1)) + ex0
    inv = 1.0 / (s + 1e-16)
    z = p * inv + b_ref[...][None, :]
    m = jnp.max(z, axis=1, keepdims=True)
    lse = jnp.log(jnp.sum(jnp.exp(z - m), axis=1, keepdims=True)) + m
    out_ref[...] = z - lse


@functools.lru_cache(maxsize=None)
def _make_sc_edge(both_halves):
    D = CLS
    mesh = plsc.VectorSubcoreMesh(core_axis_name="c", subcore_axis_name="s",
                                  num_cores=2, num_subcores=16)

    def body(edges_p, asrc_hbm, adst_hbm, h_hbm, *refs):
        if both_halves:
            pa_out, pb_out, s_out = refs[:3]
            rest = refs[3:]
        else:
            pa_out, s_out = refs[:2]
            rest = refs[2:]
        (idxb, asb, adb, rows, exc, zs, isem, absem, grsem, rsem, ssem,
         asrc_sh, adst_sh, acc_sh, s_sh, h_sh) = rest

        c = lax.axis_index("c")
        s = lax.axis_index("s")
        wid = s * 2 + c
        lane = lax.broadcasted_iota(jnp.int32, (16,), 0)

        def zero_acc():
            def _zb_zero(r, _):
                for v in range(D // 16):
                    rows[0][r, pl.ds(v * 16, 16)] = jnp.zeros((16,), jnp.float32)
                return 0
            lax.fori_loop(0, BLK, _zb_zero, 0)
            for k in range(RPS // BLK):
                pltpu.sync_copy(rows[0], acc_sh.at[pl.ds(s * RPS + k * BLK, BLK)])

        def stage_h(hoff):
            pltpu.sync_copy(h_hbm.at[pl.ds(s * RPS, RPS), pl.ds(hoff, CLS)],
                            h_sh.at[pl.ds(s * RPS, RPS)])

        zero_acc()
        for i in range(BLK // 16):
            zs[pl.ds(i * 16, 16)] = jnp.zeros((16,), jnp.float32)
        for k in range(RPS // BLK):
            pltpu.sync_copy(zs, s_sh.at[pl.ds(s * RPS + k * BLK, BLK)])
        @pl.when(s == 0)
        def _():
            pltpu.sync_copy(asrc_hbm, asrc_sh)
            pltpu.sync_copy(adst_hbm, adst_sh)
        stage_h(0)
        plsc.subcore_barrier()

        def idx_load(j, k):
            base = wid * EPT + j * BLK
            pltpu.async_copy(edges_p.at[0, pl.ds(base, BLK)], idxb[k].at[0], isem[k])
            pltpu.async_copy(edges_p.at[1, pl.ds(base, BLK)], idxb[k].at[1], isem[k])

        def gathers(j, k, first):
            base = wid * EPT + j * BLK
            pltpu.make_async_copy(edges_p.at[0, pl.ds(base, BLK)], idxb[k].at[0], isem[k]).wait()
            pltpu.make_async_copy(edges_p.at[1, pl.ds(base, BLK)], idxb[k].at[1], isem[k]).wait()
            if first:
                pltpu.async_copy(asrc_sh.at[idxb[k].at[0]], asb[k], absem[k])
                pltpu.async_copy(adst_sh.at[idxb[k].at[1]], adb[k], absem[k])
            pltpu.async_copy(h_sh.at[idxb[k].at[0]], rows[k], grsem[k])

        def drain_scatters(j, k, first):
            pltpu.make_async_copy(rows[k], acc_sh.at[idxb[k].at[1]], rsem[k]).wait()
            if first:
                pltpu.make_async_copy(exc.at[pl.ds(j * BLK, BLK)],
                                      s_sh.at[idxb[k].at[1]], ssem[k]).wait()

        def step(j, k, k1, k2, first):
            @pl.when(j >= 2)
            def _():
                drain_scatters(j - 2, k2, first)
            @pl.when(j + 2 < NB)
            def _():
                idx_load(j + 2, k2)
            @pl.when(j + 1 < NB)
            def _():
                gathers(j + 1, k1, first)

            base = wid * EPT + j * BLK
            if first:
                pltpu.make_async_copy(asrc_sh.at[idxb[k].at[0]], asb[k], absem[k]).wait()
                pltpu.make_async_copy(adst_sh.at[idxb[k].at[1]], adb[k], absem[k]).wait()
                for i in range(BLK // 16):
                    t = asb[k][pl.ds(i * 16, 16)] + adb[k][pl.ds(i * 16, 16)]
                    e = jnp.where(t > 0, t, 0.2 * t)
                    ex = jnp.exp(e)
                    gid = base + i * 16 + lane
                    ex = jnp.where(gid < E, ex, 0.0)
                    exc[pl.ds(j * BLK + i * 16, 16)] = ex
                pltpu.async_copy(exc.at[pl.ds(j * BLK, BLK)],
                                 s_sh.at[idxb[k].at[1]], ssem[k], add=True)

            pltpu.make_async_copy(h_sh.at[idxb[k].at[0]], rows[k], grsem[k]).wait()

            @plsc.parallel_loop(0, BLK, step=1, unroll=8)
            def _(i):
                m = plsc.load_gather(exc, [jnp.full((16,), j * BLK + i, jnp.int32)])
                for v in range(D // 16):
                    sl = pl.ds(v * 16, 16)
                    rows[k][i, sl] = rows[k][i, sl] * m

            pltpu.async_copy(rows[k], acc_sh.at[idxb[k].at[1]], rsem[k], add=True)

        def run_pass(first):
            idx_load(0, 0)
            idx_load(1, 1)
            gathers(0, 0, first)

            def outer(g, _):
                for k in range(DEPTH):
                    j = DEPTH * g + k
                    step(j, k, (k + 1) % DEPTH, (k + 2) % DEPTH, first)
                return 0
            lax.fori_loop(0, NB // DEPTH, outer, 0)
            drain_scatters(NB - 2, (NB - 2) % DEPTH, first)
            drain_scatters(NB - 1, (NB - 1) % DEPTH, first)
            plsc.subcore_barrier()

        def dump(part_out):
            off = s * RPS
            @pl.when(c == 0)
            def _():
                pltpu.sync_copy(acc_sh.at[pl.ds(off, RPS)],
                                part_out.at[pl.ds(off, RPS), pl.ds(0, CLS)])
            @pl.when(c == 1)
            def _():
                pltpu.sync_copy(acc_sh.at[pl.ds(off, RPS)],
                                part_out.at[pl.ds(off, RPS), pl.ds(CLS, CLS)])

        run_pass(True)
        dump(pa_out)
        off = s * RPS
        pltpu.sync_copy(s_sh.at[pl.ds(off, RPS)], s_out.at[c, pl.ds(off, RPS)])
        if both_halves:
            zero_acc()
            stage_h(CLS)
            plsc.subcore_barrier()
            run_pass(False)
            dump(pb_out)

    outs = [jax.ShapeDtypeStruct((NP, HID), jnp.float32)]
    if both_halves:
        outs.append(jax.ShapeDtypeStruct((NP, HID), jnp.float32))
    outs.append(jax.ShapeDtypeStruct((2, NP), jnp.float32))

    return pl.kernel(
        body,
        out_type=outs,
        mesh=mesh,
        scratch_types=[
            [pltpu.VMEM((2, BLK), jnp.int32) for _ in range(DEPTH)],
            [pltpu.VMEM((BLK,), jnp.float32) for _ in range(DEPTH)],
            [pltpu.VMEM((BLK,), jnp.float32) for _ in range(DEPTH)],
            [pltpu.VMEM((BLK, D), jnp.float32) for _ in range(DEPTH)],
            pltpu.VMEM((EPT,), jnp.float32),
            pltpu.VMEM((BLK,), jnp.float32),
            [pltpu.SemaphoreType.DMA for _ in range(DEPTH)],
            [pltpu.SemaphoreType.DMA for _ in range(DEPTH)],
            [pltpu.SemaphoreType.DMA for _ in range(DEPTH)],
            [pltpu.SemaphoreType.DMA for _ in range(DEPTH)],
            [pltpu.SemaphoreType.DMA for _ in range(DEPTH)],
            pltpu.VMEM_SHARED((NP,), jnp.float32),
            pltpu.VMEM_SHARED((NP,), jnp.float32),
            pltpu.VMEM_SHARED((NP, D), jnp.float32),
            pltpu.VMEM_SHARED((NP,), jnp.float32),
            pltpu.VMEM_SHARED((NP, D), jnp.float32),
        ],
        compiler_params=pltpu.CompilerParams(
            needs_layout_passes=False, use_tc_tiling_on_sc=False),
    )


_dense1 = pl.pallas_call(
    _dense1_body,
    out_shape=[
        jax.ShapeDtypeStruct((NP, HID), jnp.float32),
        jax.ShapeDtypeStruct((NP,), jnp.float32),
        jax.ShapeDtypeStruct((NP,), jnp.float32),
    ],
)

_combine2 = pl.pallas_call(
    _combine2_body,
    out_shape=[
        jax.ShapeDtypeStruct((NP, HID), jnp.float32),
        jax.ShapeDtypeStruct((NP,), jnp.float32),
        jax.ShapeDtypeStruct((NP,), jnp.float32),
    ],
)

_final = pl.pallas_call(
    _final_body,
    out_shape=jax.ShapeDtypeStruct((NP, CLS), jnp.float32),
)


def kernel(features, edges, W1, att_src1, att_dst1, b1, W2, att_src2, att_dst2, b2):
    edges_p = jnp.pad(edges, ((0, 0), (0, EP - E)))
    xp = jnp.pad(features, ((0, NP - N), (0, 0)))

    h1, a1s, a1d = _dense1(xp, W1, att_src1, att_dst1)
    pa, pb, s1 = _make_sc_edge(True)(edges_p, a1s, a1d, h1)
    h2, a2s, a2d = _combine2(pa, pb, s1, h1, a1s, a1d,
                             b1, W2, att_src2, att_dst2)
    part2, s2 = _make_sc_edge(False)(edges_p, a2s, a2d, h2)
    out = _final(part2, s2, h2, a2s, a2d, b2)
    return out[:N]
